# Initial kernel scaffold; baseline (speedup 1.0000x reference)
import jax
import jax.numpy as jnp
from jax.experimental import pallas as pl


def kernel(x, edge_index, batch, W1, b1, W2, b2, W3, b3):
    raise NotImplementedError("write your pallas kernel here")


# speedup vs baseline: 12.3713x; 12.3713x over previous
import functools

import jax
import jax.numpy as jnp
from jax import lax
from jax.experimental import pallas as pl
from jax.experimental.pallas import tpu as pltpu
from jax.experimental.pallas import tpu_sc as plsc

N = 10000
E = 320000
G = 16
IN_DIM = 128
HID = 64
EMB = 32

NC, NS = 2, 16
NW = NC * NS
CH = 128
NCHUNK = 80
EPT_PAD = NCHUNK * CH
E_PAD = EPT_PAD * NW
N_PAD = 10112
RPT = N_PAD // NS


def _make_prop(D):
    mesh = plsc.VectorSubcoreMesh(core_axis_name="c", subcore_axis_name="s")

    @functools.partial(
        pl.kernel,
        out_type=jax.ShapeDtypeStruct((NC, N_PAD, D), jnp.float32),
        mesh=mesh,
        compiler_params=pltpu.CompilerParams(use_tc_tiling_on_sc=False),
        scratch_types=[
            pltpu.VMEM_SHARED((N_PAD, D), jnp.float32),
            pltpu.VMEM((NCHUNK, CH), jnp.int32),
            pltpu.VMEM((NCHUNK, CH), jnp.int32),
            pltpu.VMEM((CH, D), jnp.float32),
            pltpu.SemaphoreType.DMA,
        ],
    )
    def prop(g_hbm, src_hbm, dst_hbm, out_hbm, acc, isrc, idst, rows, sem):
        c = lax.axis_index("c")
        s = lax.axis_index("s")
        wid = c * NS + s
        base = s * RPT
        pltpu.sync_copy(src_hbm.at[wid], isrc)
        pltpu.sync_copy(dst_hbm.at[wid], idst)
        pltpu.sync_copy(g_hbm.at[pl.ds(base, RPT)], acc.at[pl.ds(base, RPT)])
        plsc.subcore_barrier()

        def body(j, carry):
            pltpu.async_copy(g_hbm.at[isrc.at[j]], rows, sem).wait()
            pltpu.sync_copy(rows, acc.at[idst.at[j]], add=True)
            return carry

        lax.fori_loop(0, NCHUNK, body, 0)
        plsc.subcore_barrier()
        pltpu.sync_copy(acc.at[pl.ds(base, RPT)],
                        out_hbm.at[c].at[pl.ds(base, RPT)])

    return prop


_prop16 = _make_prop(16)
_prop64 = _make_prop(HID)
_prop32 = _make_prop(EMB)


def _tc_first(x_ref, w_ref, d0_ref, d1_ref, g_ref, dinv_ref):
    deg = d0_ref[...] + d1_ref[...] - 1.0
    dinv = lax.rsqrt(deg)
    dinv_ref[...] = dinv
    g_ref[...] = dinv * jnp.dot(x_ref[...], w_ref[...],
                                preferred_element_type=jnp.float32)


def _tc_mid(a0_ref, a1_ref, g_ref, dinv_ref, b_ref, w_ref, gn_ref):
    p = a0_ref[...] + a1_ref[...] - g_ref[...]
    h = jnp.maximum(dinv_ref[...] * p + b_ref[...], 0.0)
    gn_ref[...] = dinv_ref[...] * jnp.dot(h, w_ref[...],
                                          preferred_element_type=jnp.float32)


def _tc_final(a0_ref, a1_ref, g_ref, dinv_ref, b_ref, batch_ref, out_ref):
    p = a0_ref[...] + a1_ref[...] - g_ref[...]
    h = dinv_ref[...] * p + b_ref[...]
    gid = lax.broadcasted_iota(jnp.int32, (G, N_PAD), 0)
    m = (batch_ref[...] == gid).astype(jnp.float32)
    sums = jnp.dot(m, h, preferred_element_type=jnp.float32)
    cnt = jnp.sum(m, axis=1, keepdims=True)
    out_ref[...] = sums / jnp.maximum(cnt, 1.0)


def kernel(x, edge_index, batch, W1, b1, W2, b2, W3, b3):
    f32 = jnp.float32
    src = edge_index[0].astype(jnp.int32)
    dst = edge_index[1].astype(jnp.int32)
    src3 = jnp.concatenate(
        [src, jnp.zeros((E_PAD - E,), jnp.int32)]).reshape(NW, NCHUNK, CH)
    dst3 = jnp.concatenate(
        [dst, jnp.full((E_PAD - E,), N, jnp.int32)]).reshape(NW, NCHUNK, CH)

    x_p = jnp.zeros((N_PAD, IN_DIM), f32).at[:N].set(x)
    batch_p = jnp.full((1, N_PAD), G, jnp.int32).at[0, :N].set(
        batch.astype(jnp.int32))
    ones = jnp.ones((N_PAD, 16), f32)

    dparts = _prop16(ones, src3, dst3)
    d0 = lax.slice(dparts[0], (0, 0), (N_PAD, 1))
    d1 = lax.slice(dparts[1], (0, 0), (N_PAD, 1))

    g1, dinv = pl.pallas_call(
        _tc_first,
        out_shape=[jax.ShapeDtypeStruct((N_PAD, HID), f32),
                   jax.ShapeDtypeStruct((N_PAD, 1), f32)],
    )(x_p, W1, d0, d1)

    p1 = _prop64(g1, src3, dst3)
    g2 = pl.pallas_call(
        _tc_mid,
        out_shape=jax.ShapeDtypeStruct((N_PAD, HID), f32),
    )(p1[0], p1[1], g1, dinv, b1.reshape(1, HID), W2)

    p2 = _prop64(g2, src3, dst3)
    g3 = pl.pallas_call(
        _tc_mid,
        out_shape=jax.ShapeDtypeStruct((N_PAD, EMB), f32),
    )(p2[0], p2[1], g2, dinv, b2.reshape(1, HID), W3)

    p3 = _prop32(g3, src3, dst3)
    out = pl.pallas_call(
        _tc_final,
        out_shape=jax.ShapeDtypeStruct((G, EMB), f32),
    )(p3[0], p3[1], g3, dinv, b3.reshape(1, EMB), batch_p)
    return out


# speedup vs baseline: 14.6886x; 1.1873x over previous
import functools

import jax
import jax.numpy as jnp
from jax import lax
from jax.experimental import pallas as pl
from jax.experimental.pallas import tpu as pltpu
from jax.experimental.pallas import tpu_sc as plsc

N = 10000
E = 320000
G = 16
IN_DIM = 128
HID = 64
EMB = 32

NC, NS = 2, 16
NW = NC * NS
CH = 128
NCHUNK = 80
EPT_PAD = NCHUNK * CH
E_PAD = EPT_PAD * NW
N_PAD = 10112
RPT = N_PAD // NS


def _make_prop(D):
    mesh = plsc.VectorSubcoreMesh(core_axis_name="c", subcore_axis_name="s")

    @functools.partial(
        pl.kernel,
        out_type=jax.ShapeDtypeStruct((NC, N_PAD, D), jnp.float32),
        mesh=mesh,
        compiler_params=pltpu.CompilerParams(use_tc_tiling_on_sc=False),
        scratch_types=[
            pltpu.VMEM_SHARED((N_PAD, D), jnp.float32),
            pltpu.VMEM((NCHUNK, CH), jnp.int32),
            pltpu.VMEM((NCHUNK, CH), jnp.int32),
            pltpu.VMEM((CH, D), jnp.float32),
            pltpu.VMEM((CH, D), jnp.float32),
            pltpu.SemaphoreType.DMA,
            pltpu.SemaphoreType.DMA,
            pltpu.SemaphoreType.DMA,
            pltpu.SemaphoreType.DMA,
        ],
    )
    def prop(g_hbm, src_hbm, dst_hbm, out_hbm, acc, isrc, idst,
             rows0, rows1, gs0, gs1, ss0, ss1):
        c = lax.axis_index("c")
        s = lax.axis_index("s")
        wid = c * NS + s
        base = s * RPT
        pltpu.sync_copy(src_hbm.at[wid], isrc)
        pltpu.sync_copy(dst_hbm.at[wid], idst)
        pltpu.sync_copy(g_hbm.at[pl.ds(base, RPT)], acc.at[pl.ds(base, RPT)])
        plsc.subcore_barrier()

        pltpu.async_copy(g_hbm.at[isrc.at[0]], rows0, gs0)
        pltpu.async_copy(g_hbm.at[isrc.at[1]], rows1, gs1)

        def body(jj, carry):
            j0 = jj * 2
            j1 = j0 + 1
            pltpu.make_async_copy(g_hbm.at[isrc.at[j0]], rows0, gs0).wait()
            pltpu.async_copy(rows0, acc.at[idst.at[j0]], ss0, add=True)
            pltpu.make_async_copy(g_hbm.at[isrc.at[j1]], rows1, gs1).wait()
            pltpu.async_copy(rows1, acc.at[idst.at[j1]], ss1, add=True)

            @pl.when(jj < NCHUNK // 2 - 1)
            def _():
                pltpu.make_async_copy(
                    rows0, acc.at[idst.at[j0]], ss0).wait()
                pltpu.async_copy(g_hbm.at[isrc.at[j0 + 2]], rows0, gs0)
                pltpu.make_async_copy(
                    rows1, acc.at[idst.at[j1]], ss1).wait()
                pltpu.async_copy(g_hbm.at[isrc.at[j1 + 2]], rows1, gs1)

            @pl.when(jj == NCHUNK // 2 - 1)
            def _():
                pltpu.make_async_copy(
                    rows0, acc.at[idst.at[j0]], ss0).wait()
                pltpu.make_async_copy(
                    rows1, acc.at[idst.at[j1]], ss1).wait()
            return carry

        lax.fori_loop(0, NCHUNK // 2, body, 0)
        plsc.subcore_barrier()
        pltpu.sync_copy(acc.at[pl.ds(base, RPT)],
                        out_hbm.at[c].at[pl.ds(base, RPT)])

    return prop


_prop64 = _make_prop(HID)
_prop32 = _make_prop(EMB)

DEGW = 16


def _make_deg():
    mesh = plsc.VectorSubcoreMesh(core_axis_name="c", subcore_axis_name="s")

    @functools.partial(
        pl.kernel,
        out_type=jax.ShapeDtypeStruct((NC, N_PAD, DEGW), jnp.float32),
        mesh=mesh,
        compiler_params=pltpu.CompilerParams(use_tc_tiling_on_sc=False),
        scratch_types=[
            pltpu.VMEM_SHARED((N_PAD, DEGW), jnp.float32),
            pltpu.VMEM((NCHUNK, CH), jnp.int32),
            pltpu.VMEM((CH, DEGW), jnp.float32),
            pltpu.SemaphoreType.DMA,
        ],
    )
    def deg(ones_hbm, dst_hbm, out_hbm, acc, idst, rows, sem):
        c = lax.axis_index("c")
        s = lax.axis_index("s")
        wid = c * NS + s
        base = s * RPT
        pltpu.sync_copy(dst_hbm.at[wid], idst)
        pltpu.sync_copy(ones_hbm.at[pl.ds(0, CH)], rows)
        pltpu.sync_copy(ones_hbm.at[pl.ds(base, RPT)],
                        acc.at[pl.ds(base, RPT)])
        plsc.subcore_barrier()

        def body(jo, carry):
            for u in range(8):
                pltpu.async_copy(rows, acc.at[idst.at[jo * 8 + u]], sem,
                                 add=True)
            for u in range(8):
                pltpu.make_async_copy(rows, acc.at[idst.at[jo * 8 + u]],
                                      sem).wait()
            return carry

        lax.fori_loop(0, NCHUNK // 8, body, 0)
        plsc.subcore_barrier()
        pltpu.sync_copy(acc.at[pl.ds(base, RPT)],
                        out_hbm.at[c].at[pl.ds(base, RPT)])

    return deg


_deg = _make_deg()


def _tc_first(x_ref, w_ref, d0_ref, d1_ref, g_ref, dinv_ref):
    deg = d0_ref[...] + d1_ref[...] - 1.0
    dinv = lax.rsqrt(deg)
    dinv_ref[...] = dinv
    g_ref[...] = dinv * jnp.dot(x_ref[...], w_ref[...],
                                preferred_element_type=jnp.float32)


def _tc_mid(a0_ref, a1_ref, g_ref, dinv_ref, b_ref, w_ref, gn_ref):
    p = a0_ref[...] + a1_ref[...] - g_ref[...]
    h = jnp.maximum(dinv_ref[...] * p + b_ref[...], 0.0)
    gn_ref[...] = dinv_ref[...] * jnp.dot(h, w_ref[...],
                                          preferred_element_type=jnp.float32)


def _tc_final(a0_ref, a1_ref, g_ref, dinv_ref, b_ref, batch_ref, out_ref):
    p = a0_ref[...] + a1_ref[...] - g_ref[...]
    h = dinv_ref[...] * p + b_ref[...]
    gid = lax.broadcasted_iota(jnp.int32, (G, N_PAD), 0)
    m = (batch_ref[...] == gid).astype(jnp.float32)
    sums = jnp.dot(m, h, preferred_element_type=jnp.float32)
    cnt = jnp.sum(m, axis=1, keepdims=True)
    out_ref[...] = sums / jnp.maximum(cnt, 1.0)


def kernel(x, edge_index, batch, W1, b1, W2, b2, W3, b3):
    f32 = jnp.float32
    src = edge_index[0].astype(jnp.int32)
    dst = edge_index[1].astype(jnp.int32)
    src3 = jnp.concatenate(
        [src, jnp.zeros((E_PAD - E,), jnp.int32)]).reshape(NW, NCHUNK, CH)
    dst3 = jnp.concatenate(
        [dst, jnp.full((E_PAD - E,), N, jnp.int32)]).reshape(NW, NCHUNK, CH)

    x_p = jnp.zeros((N_PAD, IN_DIM), f32).at[:N].set(x)
    batch_p = jnp.full((1, N_PAD), G, jnp.int32).at[0, :N].set(
        batch.astype(jnp.int32))
    ones = jnp.ones((N_PAD, DEGW), f32)

    dparts = _deg(ones, dst3)
    d0 = lax.slice(dparts[0], (0, 0), (N_PAD, 1))
    d1 = lax.slice(dparts[1], (0, 0), (N_PAD, 1))

    g1, dinv = pl.pallas_call(
        _tc_first,
        out_shape=[jax.ShapeDtypeStruct((N_PAD, HID), f32),
                   jax.ShapeDtypeStruct((N_PAD, 1), f32)],
    )(x_p, W1, d0, d1)

    p1 = _prop64(g1, src3, dst3)
    g2 = pl.pallas_call(
        _tc_mid,
        out_shape=jax.ShapeDtypeStruct((N_PAD, HID), f32),
    )(p1[0], p1[1], g1, dinv, b1.reshape(1, HID), W2)

    p2 = _prop64(g2, src3, dst3)
    g3 = pl.pallas_call(
        _tc_mid,
        out_shape=jax.ShapeDtypeStruct((N_PAD, EMB), f32),
    )(p2[0], p2[1], g2, dinv, b2.reshape(1, HID), W3)

    p3 = _prop32(g3, src3, dst3)
    out = pl.pallas_call(
        _tc_final,
        out_shape=jax.ShapeDtypeStruct((G, EMB), f32),
    )(p3[0], p3[1], g3, dinv, b3.reshape(1, EMB), batch_p)
    return out
